# baseline (speedup 1.0000x reference)
import functools

import jax
import jax.numpy as jnp
from jax import lax
from jax.experimental import pallas as pl
from jax.experimental.pallas import tpu as pltpu
from jax.experimental.pallas import tpu_sc as plsc

_DIM = 2048
_BATCH = 128
_NC = 2
_NW = 16
_BPW = _BATCH // _NW
_H = _BPW // 2

_mesh = plsc.VectorSubcoreMesh(core_axis_name="c", subcore_axis_name="s")


@functools.partial(
    pl.kernel,
    mesh=_mesh,
    out_type=jax.ShapeDtypeStruct((_BATCH, 1, _DIM), jnp.float32),
    scratch_types=[
        pltpu.VMEM((2, _H), jnp.int32),
        pltpu.VMEM((_H, 1, _DIM), jnp.float32),
        pltpu.VMEM((_H, 1, _DIM), jnp.float32),
        pltpu.SemaphoreType.DMA,
        pltpu.SemaphoreType.DMA,
    ],
)
def _embed(idx_hbm, table_hbm, out_hbm, idx_v, rows_a, rows_b, sem_a, sem_b):
    wid = lax.axis_index("s") * _NC + lax.axis_index("c")

    @pl.when(wid < _NW)
    def _():
        base = wid * _BPW
        pltpu.sync_copy(idx_hbm.at[wid], idx_v)
        cp_a = pltpu.async_copy(table_hbm.at[idx_v.at[0]], rows_a, sem_a)
        cp_b = pltpu.async_copy(table_hbm.at[idx_v.at[1]], rows_b, sem_b)
        cp_a.wait()
        pltpu.sync_copy(rows_a, out_hbm.at[pl.ds(base, _H)])
        cp_b.wait()
        pltpu.sync_copy(rows_b, out_hbm.at[pl.ds(base + _H, _H)])


def kernel(stage_id, weight):
    idx3d = stage_id.astype(jnp.int32).reshape(_NW, 2, _H)
    return _embed(idx3d, weight.reshape(3, 1, _DIM))


# speedup vs baseline: 1.0788x; 1.0788x over previous
import functools

import jax
import jax.numpy as jnp
from jax import lax
from jax.experimental import pallas as pl
from jax.experimental.pallas import tpu as pltpu
from jax.experimental.pallas import tpu_sc as plsc

_DIM = 2048
_BATCH = 128
_NC = 2
_NW = 16
_BPW = _BATCH // _NW

_mesh = plsc.VectorSubcoreMesh(core_axis_name="c", subcore_axis_name="s")


@functools.partial(
    pl.kernel,
    mesh=_mesh,
    out_type=jax.ShapeDtypeStruct((_BATCH, 1, _DIM), jnp.float32),
    scratch_types=[
        pltpu.VMEM((_BPW,), jnp.int32),
        pltpu.VMEM((_BPW, 1, _DIM), jnp.float32),
        pltpu.SemaphoreType.DMA,
    ],
    compiler_params=pltpu.CompilerParams(allow_input_fusion=[True, True]),
)
def _embed(idx_hbm, table_hbm, out_hbm, idx_v, rows_v, sem):
    wid = lax.axis_index("s") * _NC + lax.axis_index("c")

    @pl.when(wid < _NW)
    def _():
        base = wid * _BPW
        pltpu.sync_copy(idx_hbm.at[pl.ds(base, _BPW)], idx_v)
        pltpu.async_copy(table_hbm.at[idx_v], rows_v, sem).wait()
        pltpu.sync_copy(rows_v, out_hbm.at[pl.ds(base, _BPW)])


def kernel(stage_id, weight):
    return _embed(stage_id.astype(jnp.int32), weight.reshape(3, 1, _DIM))


# speedup vs baseline: 1.1353x; 1.0524x over previous
import functools

import jax
import jax.numpy as jnp
from jax import lax
from jax.experimental import pallas as pl
from jax.experimental.pallas import tpu as pltpu
from jax.experimental.pallas import tpu_sc as plsc

_DIM = 2048
_BATCH = 128
_STAGES = 3
_NC = 2
_NW = 16
_BPW = _BATCH // _NW

_mesh = plsc.VectorSubcoreMesh(core_axis_name="c", subcore_axis_name="s")


@functools.partial(
    pl.kernel,
    mesh=_mesh,
    out_type=jax.ShapeDtypeStruct((_BATCH, 1, _DIM), jnp.float32),
    scratch_types=[
        pltpu.VMEM((_BPW,), jnp.int32),
        pltpu.VMEM_SHARED((_STAGES, 1, _DIM), jnp.float32),
        pltpu.VMEM((_BPW, 1, _DIM), jnp.float32),
        pltpu.SemaphoreType.DMA,
        pltpu.SemaphoreType.DMA,
    ],
)
def _embed(idx_hbm, table_hbm, out_hbm, idx_v, table_sh, rows_v, sem_t, sem_g):
    wid = lax.axis_index("s") * _NC + lax.axis_index("c")
    sid = lax.axis_index("s")

    @pl.when(sid == 0)
    def _():
        pltpu.async_copy(table_hbm, table_sh, sem_t).wait()

    @pl.when(wid < _NW)
    def _():
        base = wid * _BPW
        pltpu.sync_copy(idx_hbm.at[pl.ds(base, _BPW)], idx_v)

    plsc.subcore_barrier()

    @pl.when(wid < _NW)
    def _():
        base = wid * _BPW
        pltpu.async_copy(table_sh.at[idx_v], rows_v, sem_g).wait()
        pltpu.sync_copy(rows_v, out_hbm.at[pl.ds(base, _BPW)])


def kernel(stage_id, weight):
    return _embed(stage_id.astype(jnp.int32), weight.reshape(_STAGES, 1, _DIM))


# speedup vs baseline: 1.1562x; 1.0185x over previous
import functools

import jax
import jax.numpy as jnp
from jax import lax
from jax.experimental import pallas as pl
from jax.experimental.pallas import tpu as pltpu
from jax.experimental.pallas import tpu_sc as plsc

_DIM = 2048
_BATCH = 128
_STAGES = 3
_NC = 2
_NW = 16
_BPW = _BATCH // _NW

_mesh = plsc.VectorSubcoreMesh(core_axis_name="c", subcore_axis_name="s")


@functools.partial(
    pl.kernel,
    mesh=_mesh,
    out_type=jax.ShapeDtypeStruct((_BATCH, 1, _DIM), jnp.float32),
    scratch_types=[
        pltpu.VMEM((_BPW,), jnp.int32),
        pltpu.VMEM_SHARED((_STAGES, 1, _DIM), jnp.float32),
        pltpu.VMEM((_BPW, 1, _DIM), jnp.float32),
        pltpu.SemaphoreType.DMA,
        pltpu.SemaphoreType.DMA,
    ],
)
def _embed(idx_hbm, table_hbm, out_hbm, idx_v, table_sh, rows_v, sem_t, sem_g):
    wid = lax.axis_index("s") * _NC + lax.axis_index("c")
    sid = lax.axis_index("s")

    @pl.when(sid == 8)
    def _():
        pltpu.async_copy(table_hbm, table_sh, sem_t).wait()

    @pl.when(wid < _NW)
    def _():
        base = wid * _BPW
        pltpu.sync_copy(idx_hbm.at[pl.ds(base, _BPW)], idx_v)

    plsc.subcore_barrier()

    @pl.when(wid < _NW)
    def _():
        base = wid * _BPW
        pltpu.async_copy(table_sh.at[idx_v], rows_v, sem_g).wait()
        pltpu.sync_copy(rows_v, out_hbm.at[pl.ds(base, _BPW)])


def kernel(stage_id, weight):
    return _embed(stage_id.astype(jnp.int32), weight.reshape(_STAGES, 1, _DIM))
